# baseline (speedup 1.0000x reference)
import functools

import jax
import jax.numpy as jnp
from jax import lax
from jax.experimental import pallas as pl
from jax.experimental.pallas import tpu as pltpu
from jax.experimental.pallas import tpu_sc as plsc

VOCAB_SIZE = 100000
DIM = 300
LANES = 16
BLK = 128
CHUNK = 80

PDIM = 3 * BLK

_ZERO_STARTS = tuple(range(0, DIM + 4, LANES))


def _make_kernel(num_tokens):
    info = plsc.get_sparse_core_info()
    num_workers = info.num_cores * info.num_subcores
    per_worker = num_tokens // num_workers
    num_chunks = per_worker // CHUNK
    mesh = plsc.VectorSubcoreMesh(core_axis_name="c", subcore_axis_name="s")

    @functools.partial(
        pl.kernel,
        mesh=mesh,
        out_type=jax.ShapeDtypeStruct((num_tokens, PDIM), jnp.float32),
        scratch_types=[
            pltpu.VMEM((per_worker + LANES,), jnp.int32),
            pltpu.VMEM((CHUNK,), jnp.int32),
            pltpu.VMEM((CHUNK,), jnp.int32),
            pltpu.VMEM((CHUNK, BLK), jnp.float32),
            pltpu.VMEM((CHUNK, BLK), jnp.float32),
            pltpu.VMEM((CHUNK, PDIM), jnp.float32),
            pltpu.VMEM((CHUNK, PDIM), jnp.float32),
            pltpu.SemaphoreType.DMA,
            pltpu.SemaphoreType.DMA,
            pltpu.SemaphoreType.DMA,
        ],
    )
    def emb_kernel(table_hbm, side_hbm, idx_hbm, out_hbm, ids_v, gidx0,
                   gidx1, ws0, ws1, pk0, pk1, sem_g, sem_o0, sem_o1):
        wid = lax.axis_index("s") * info.num_cores + lax.axis_index("c")
        base = wid * per_worker

        zeros16 = jnp.zeros((LANES,), jnp.float32)

        pltpu.sync_copy(idx_hbm.at[pl.ds(base, per_worker)],
                        ids_v.at[pl.ds(0, per_worker)])

        def build(c, gidx_ref):
            off = c * CHUNK
            for grp in range(CHUNK // LANES):
                v = ids_v[pl.ds(off + grp * LANES, LANES)]
                gidx_ref[pl.ds(grp * LANES, LANES)] = jnp.minimum(
                    v, VOCAB_SIZE - 1)

        def start_gathers(gidx_ref, ws_ref, pk_ref):
            h1 = pltpu.async_copy(
                table_hbm.at[gidx_ref, pl.ds(0, BLK)],
                pk_ref.at[:, pl.ds(0, BLK)], sem_g)
            h2 = pltpu.async_copy(
                table_hbm.at[gidx_ref, pl.ds(BLK, BLK)],
                pk_ref.at[:, pl.ds(BLK, BLK)], sem_g)
            h3 = pltpu.async_copy(side_hbm.at[gidx_ref], ws_ref, sem_g)
            return h1, h2, h3

        def compact(c, ws_ref, pk_ref):
            off = c * CHUNK

            def one(t, _):
                v = ids_v[pl.ds(off + t, LANES)][0]

                @pl.when(v < VOCAB_SIZE)
                def _():
                    pk_ref[t, pl.ds(2 * BLK, LANES)] = ws_ref[t, pl.ds(0, LANES)]
                    pk_ref[t, pl.ds(2 * BLK + LANES, LANES)] = (
                        ws_ref[t, pl.ds(LANES, LANES)])
                    pk_ref[t, pl.ds(2 * BLK + 2 * LANES, LANES)] = (
                        ws_ref[t, pl.ds(2 * LANES, LANES)])

                @pl.when(v >= VOCAB_SIZE)
                def _():
                    for o in _ZERO_STARTS:
                        pk_ref[t, pl.ds(o, LANES)] = zeros16

                return 0

            lax.fori_loop(0, CHUNK, one, 0)

        def start_out(c, pk_ref, sem):
            return pltpu.async_copy(
                pk_ref, out_hbm.at[pl.ds(base + c * CHUNK, CHUNK)], sem)

        def wait_out(c, pk_ref, sem):
            pltpu.make_async_copy(
                pk_ref, out_hbm.at[pl.ds(base + c * CHUNK, CHUNK)], sem).wait()

        def round_(c, gidx_ref, ws_ref, pk_ref, sem_o, ws_prev, pk_prev,
                   sem_o_prev):
            build(c, gidx_ref)

            @pl.when(c >= 2)
            def _():
                wait_out(c - 2, pk_ref, sem_o)

            hs = start_gathers(gidx_ref, ws_ref, pk_ref)

            @pl.when(c >= 1)
            def _():
                compact(c - 1, ws_prev, pk_prev)
                start_out(c - 1, pk_prev, sem_o_prev)

            for h in hs:
                h.wait()

        def loop_body(c, _):
            @pl.when((c & 1) == 0)
            def _():
                round_(c, gidx0, ws0, pk0, sem_o0, ws1, pk1, sem_o1)

            @pl.when((c & 1) == 1)
            def _():
                round_(c, gidx1, ws1, pk1, sem_o1, ws0, pk0, sem_o0)

            return 0

        lax.fori_loop(0, num_chunks, loop_body, 0)

        last = num_chunks - 1
        compact(last, ws1, pk1)
        start_out(last, pk1, sem_o1)
        wait_out(last - 1, pk0, sem_o0)
        wait_out(last, pk1, sem_o1)

    return emb_kernel


@jax.jit
def kernel(indices, table):
    batch, seq = indices.shape
    side = jnp.pad(table[:, 2 * BLK:], ((0, 0), (0, 3 * BLK - DIM)))
    flat_idx = indices.reshape(batch * seq)
    out = _make_kernel(batch * seq)(table, side, flat_idx)
    return out[:, :DIM].reshape(batch, seq, DIM)


# speedup vs baseline: 1.0006x; 1.0006x over previous
import functools

import jax
import jax.numpy as jnp
from jax import lax
from jax.experimental import pallas as pl
from jax.experimental.pallas import tpu as pltpu
from jax.experimental.pallas import tpu_sc as plsc

VOCAB_SIZE = 100000
DIM = 300
LANES = 16
BLK = 128
CHUNK = 80

PDIM = 304

_ZERO_STARTS = tuple(range(0, DIM + 4, LANES))


def _make_kernel(num_tokens):
    info = plsc.get_sparse_core_info()
    num_workers = info.num_cores * info.num_subcores
    per_worker = num_tokens // num_workers
    num_chunks = per_worker // CHUNK
    mesh = plsc.VectorSubcoreMesh(core_axis_name="c", subcore_axis_name="s")

    @functools.partial(
        pl.kernel,
        mesh=mesh,
        out_type=jax.ShapeDtypeStruct((num_tokens, PDIM), jnp.float32),
        scratch_types=[
            pltpu.VMEM((per_worker + LANES,), jnp.int32),
            pltpu.VMEM((CHUNK,), jnp.int32),
            pltpu.VMEM((CHUNK,), jnp.int32),
            pltpu.VMEM((CHUNK, BLK), jnp.float32),
            pltpu.VMEM((CHUNK, BLK), jnp.float32),
            pltpu.VMEM((CHUNK, PDIM), jnp.float32),
            pltpu.VMEM((CHUNK, PDIM), jnp.float32),
            pltpu.SemaphoreType.DMA,
            pltpu.SemaphoreType.DMA,
            pltpu.SemaphoreType.DMA,
        ],
    )
    def emb_kernel(table_hbm, side_hbm, idx_hbm, out_hbm, ids_v, gidx0,
                   gidx1, ws0, ws1, pk0, pk1, sem_g, sem_o0, sem_o1):
        wid = lax.axis_index("s") * info.num_cores + lax.axis_index("c")
        base = wid * per_worker

        zeros16 = jnp.zeros((LANES,), jnp.float32)

        pltpu.sync_copy(idx_hbm.at[pl.ds(base, per_worker)],
                        ids_v.at[pl.ds(0, per_worker)])

        def build(c, gidx_ref):
            off = c * CHUNK
            for grp in range(CHUNK // LANES):
                v = ids_v[pl.ds(off + grp * LANES, LANES)]
                gidx_ref[pl.ds(grp * LANES, LANES)] = jnp.minimum(
                    v, VOCAB_SIZE - 1)

        def start_gathers(gidx_ref, ws_ref, pk_ref):
            h1 = pltpu.async_copy(
                table_hbm.at[gidx_ref, pl.ds(0, 2 * BLK)],
                pk_ref.at[:, pl.ds(0, 2 * BLK)], sem_g)
            h3 = pltpu.async_copy(side_hbm.at[gidx_ref], ws_ref, sem_g)
            return h1, h3

        def compact(c, ws_ref, pk_ref):
            off = c * CHUNK

            def one(t, _):
                v = ids_v[pl.ds(off + t, LANES)][0]

                @pl.when(v < VOCAB_SIZE)
                def _():
                    pk_ref[t, pl.ds(2 * BLK, LANES)] = ws_ref[t, pl.ds(0, LANES)]
                    pk_ref[t, pl.ds(2 * BLK + LANES, LANES)] = (
                        ws_ref[t, pl.ds(LANES, LANES)])
                    pk_ref[t, pl.ds(2 * BLK + 2 * LANES, LANES)] = (
                        ws_ref[t, pl.ds(2 * LANES, LANES)])

                @pl.when(v >= VOCAB_SIZE)
                def _():
                    for o in _ZERO_STARTS:
                        pk_ref[t, pl.ds(o, LANES)] = zeros16

                return 0

            lax.fori_loop(0, CHUNK, one, 0)

        def start_out(c, pk_ref, sem):
            return pltpu.async_copy(
                pk_ref, out_hbm.at[pl.ds(base + c * CHUNK, CHUNK)], sem)

        def wait_out(c, pk_ref, sem):
            pltpu.make_async_copy(
                pk_ref, out_hbm.at[pl.ds(base + c * CHUNK, CHUNK)], sem).wait()

        def round_(c, gidx_ref, ws_ref, pk_ref, sem_o, ws_prev, pk_prev,
                   sem_o_prev):
            build(c, gidx_ref)

            @pl.when(c >= 2)
            def _():
                wait_out(c - 2, pk_ref, sem_o)

            hs = start_gathers(gidx_ref, ws_ref, pk_ref)

            @pl.when(c >= 1)
            def _():
                compact(c - 1, ws_prev, pk_prev)
                start_out(c - 1, pk_prev, sem_o_prev)

            for h in hs:
                h.wait()

        def loop_body(c, _):
            @pl.when((c & 1) == 0)
            def _():
                round_(c, gidx0, ws0, pk0, sem_o0, ws1, pk1, sem_o1)

            @pl.when((c & 1) == 1)
            def _():
                round_(c, gidx1, ws1, pk1, sem_o1, ws0, pk0, sem_o0)

            return 0

        lax.fori_loop(0, num_chunks, loop_body, 0)

        last = num_chunks - 1
        compact(last, ws1, pk1)
        start_out(last, pk1, sem_o1)
        wait_out(last - 1, pk0, sem_o0)
        wait_out(last, pk1, sem_o1)

    return emb_kernel


@jax.jit
def kernel(indices, table):
    batch, seq = indices.shape
    side = jnp.pad(table[:, 2 * BLK:], ((0, 0), (0, 3 * BLK - DIM)))
    flat_idx = indices.reshape(batch * seq)
    out = _make_kernel(batch * seq)(table, side, flat_idx)
    return out[:, :DIM].reshape(batch, seq, DIM)


# speedup vs baseline: 1.0019x; 1.0013x over previous
import functools

import jax
import jax.numpy as jnp
from jax import lax
from jax.experimental import pallas as pl
from jax.experimental.pallas import tpu as pltpu
from jax.experimental.pallas import tpu_sc as plsc

VOCAB_SIZE = 100000
DIM = 300
LANES = 16
BLK = 128
CHUNK = 80

PDIM = 304

_ZERO_STARTS = tuple(range(0, DIM + 4, LANES))


def _make_kernel(num_tokens):
    info = plsc.get_sparse_core_info()
    num_workers = info.num_cores * info.num_subcores
    per_worker = num_tokens // num_workers
    num_chunks = per_worker // CHUNK
    mesh = plsc.VectorSubcoreMesh(core_axis_name="c", subcore_axis_name="s")

    @functools.partial(
        pl.kernel,
        mesh=mesh,
        compiler_params=pltpu.CompilerParams(needs_layout_passes=False),
        out_type=jax.ShapeDtypeStruct((num_tokens, PDIM), jnp.float32),
        scratch_types=[
            pltpu.VMEM((per_worker + LANES,), jnp.int32),
            pltpu.VMEM((CHUNK,), jnp.int32),
            pltpu.VMEM((CHUNK,), jnp.int32),
            pltpu.VMEM((CHUNK, BLK), jnp.float32),
            pltpu.VMEM((CHUNK, BLK), jnp.float32),
            pltpu.VMEM((CHUNK, PDIM), jnp.float32),
            pltpu.VMEM((CHUNK, PDIM), jnp.float32),
            pltpu.SemaphoreType.DMA,
            pltpu.SemaphoreType.DMA,
            pltpu.SemaphoreType.DMA,
        ],
    )
    def emb_kernel(table_hbm, side_hbm, idx_hbm, out_hbm, ids_v, gidx0,
                   gidx1, ws0, ws1, pk0, pk1, sem_g, sem_o0, sem_o1):
        wid = lax.axis_index("s") * info.num_cores + lax.axis_index("c")
        base = wid * per_worker

        zeros16 = jnp.zeros((LANES,), jnp.float32)

        pltpu.sync_copy(idx_hbm.at[pl.ds(base, per_worker)],
                        ids_v.at[pl.ds(0, per_worker)])

        def build(c, gidx_ref):
            off = c * CHUNK
            for grp in range(CHUNK // LANES):
                v = ids_v[pl.ds(off + grp * LANES, LANES)]
                gidx_ref[pl.ds(grp * LANES, LANES)] = jnp.minimum(
                    v, VOCAB_SIZE - 1)

        def start_gathers(gidx_ref, ws_ref, pk_ref):
            h1 = pltpu.async_copy(
                table_hbm.at[gidx_ref, pl.ds(0, 2 * BLK)],
                pk_ref.at[:, pl.ds(0, 2 * BLK)], sem_g)
            h3 = pltpu.async_copy(side_hbm.at[gidx_ref], ws_ref, sem_g)
            return h1, h3

        def compact(c, ws_ref, pk_ref):
            off = c * CHUNK

            def one(t, _):
                pk_ref[t, pl.ds(2 * BLK, LANES)] = ws_ref[t, pl.ds(0, LANES)]
                pk_ref[t, pl.ds(2 * BLK + LANES, LANES)] = (
                    ws_ref[t, pl.ds(LANES, LANES)])
                pk_ref[t, pl.ds(2 * BLK + 2 * LANES, LANES)] = (
                    ws_ref[t, pl.ds(2 * LANES, LANES)])
                return 0

            lax.fori_loop(0, CHUNK, one, 0)

            def grp(g, _):
                v = ids_v[pl.ds(off + g * LANES, LANES)]
                n_oov = plsc.all_reduce_population_count(v >= VOCAB_SIZE)

                @pl.when(n_oov[0] > 0)
                def _():
                    def zero_one(t, _):
                        vv = ids_v[pl.ds(off + g * LANES + t, LANES)][0]

                        @pl.when(vv >= VOCAB_SIZE)
                        def _():
                            for o in _ZERO_STARTS:
                                pk_ref[g * LANES + t, pl.ds(o, LANES)] = zeros16

                        return 0

                    lax.fori_loop(0, LANES, zero_one, 0)

                return 0

            lax.fori_loop(0, CHUNK // LANES, grp, 0)

        def start_out(c, pk_ref, sem):
            return pltpu.async_copy(
                pk_ref, out_hbm.at[pl.ds(base + c * CHUNK, CHUNK)], sem)

        def wait_out(c, pk_ref, sem):
            pltpu.make_async_copy(
                pk_ref, out_hbm.at[pl.ds(base + c * CHUNK, CHUNK)], sem).wait()

        def round_(c, gidx_ref, ws_ref, pk_ref, sem_o, ws_prev, pk_prev,
                   sem_o_prev):
            build(c, gidx_ref)

            @pl.when(c >= 2)
            def _():
                wait_out(c - 2, pk_ref, sem_o)

            hs = start_gathers(gidx_ref, ws_ref, pk_ref)

            @pl.when(c >= 1)
            def _():
                compact(c - 1, ws_prev, pk_prev)
                start_out(c - 1, pk_prev, sem_o_prev)

            for h in hs:
                h.wait()

        def loop_body(c, _):
            @pl.when((c & 1) == 0)
            def _():
                round_(c, gidx0, ws0, pk0, sem_o0, ws1, pk1, sem_o1)

            @pl.when((c & 1) == 1)
            def _():
                round_(c, gidx1, ws1, pk1, sem_o1, ws0, pk0, sem_o0)

            return 0

        lax.fori_loop(0, num_chunks, loop_body, 0)

        last = num_chunks - 1
        compact(last, ws1, pk1)
        start_out(last, pk1, sem_o1)
        wait_out(last - 1, pk0, sem_o0)
        wait_out(last, pk1, sem_o1)

    return emb_kernel


@jax.jit
def kernel(indices, table):
    batch, seq = indices.shape
    side = jnp.pad(table[:, 2 * BLK:], ((0, 0), (0, 3 * BLK - DIM)))
    flat_idx = indices.reshape(batch * seq)
    out = _make_kernel(batch * seq)(table, side, flat_idx)
    return out[:, :DIM].reshape(batch, seq, DIM)
